# Initial kernel scaffold; baseline (speedup 1.0000x reference)
import jax
import jax.numpy as jnp
from jax.experimental import pallas as pl


def kernel(x, edge_index, conv_W0, conv_b0, conv_W1, conv_b1, conv_W2, conv_b2, ln_g0, ln_b0, ln_g1, ln_b1, mlp_W0, mlp_b0, mlp_W1, mlp_b1):
    raise NotImplementedError("write your pallas kernel here")


# speedup vs baseline: 51.7356x; 51.7356x over previous
import jax
import jax.numpy as jnp
from jax import lax
from jax.scipy.linalg import block_diag
from jax.experimental import pallas as pl
from jax.experimental.pallas import tpu as pltpu
from jax.experimental.pallas import tpu_sc as plsc

N = 10000
E = 320000
DIN, DH, DOUT = 128, 32, 40
NC, NS = 2, 16
NW = NC * NS
CH = 128
K = 80
EPW = CH * K
E_PAD = NW * EPW
N_PAD = 10240
RPT = N_PAD // NS
NPK = N_PAD // 4

_mesh = plsc.VectorSubcoreMesh(
    core_axis_name="c", subcore_axis_name="s", num_cores=NC, num_subcores=NS
)
_sc_params = pltpu.CompilerParams(use_tc_tiling_on_sc=False)


def _deg_body(dst_hbm, ones_hbm, zeros_hbm, out_hbm, dstv, onesv, stage, acc):
    c = lax.axis_index("c")
    s = lax.axis_index("s")
    w = s * NC + c
    pltpu.sync_copy(zeros_hbm.at[pl.ds(s * RPT, RPT)], stage)
    pltpu.sync_copy(stage, acc.at[pl.ds(s * RPT, RPT)])
    pltpu.sync_copy(dst_hbm.at[pl.ds(w * K, K)], dstv)
    pltpu.sync_copy(ones_hbm, onesv)
    plsc.subcore_barrier()

    @pl.loop(0, K)
    def _(j):
        pltpu.sync_copy(onesv, acc.at[dstv.at[j]], add=True)

    plsc.subcore_barrier()
    pltpu.sync_copy(acc.at[pl.ds(s * RPT, RPT)], stage)
    pltpu.sync_copy(stage, out_hbm.at[c, pl.ds(s * RPT, RPT)])


_deg_call = pl.kernel(
    _deg_body,
    out_type=jax.ShapeDtypeStruct((NC, N_PAD, DH), jnp.float32),
    mesh=_mesh,
    compiler_params=_sc_params,
    scratch_types=[
        pltpu.VMEM((K, CH), jnp.int32),
        pltpu.VMEM((CH, DH), jnp.float32),
        pltpu.VMEM((RPT, DH), jnp.float32),
        pltpu.VMEM_SHARED((N_PAD, DH), jnp.float32),
    ],
)


B = 4
NSETS = 2
ROUNDS = K // (NSETS * B)


def _msg_body(g_hbm, src_hbm, dst_hbm, zeros_hbm, out_hbm, srcv, dstv, bufs, stage,
              acc, gsh, gsem, ssem):
    c = lax.axis_index("c")
    s = lax.axis_index("s")
    w = s * NC + c
    pltpu.sync_copy(zeros_hbm.at[pl.ds(s * RPT, RPT)], stage)
    pltpu.sync_copy(stage, acc.at[pl.ds(s * RPT, RPT)])
    pltpu.sync_copy(g_hbm.at[pl.ds(s * RPT, RPT)], stage)
    pltpu.sync_copy(stage, gsh.at[pl.ds(s * RPT, RPT)])
    pltpu.sync_copy(src_hbm.at[pl.ds(w * K, K)], srcv)
    pltpu.sync_copy(dst_hbm.at[pl.ds(w * K, K)], dstv)
    plsc.subcore_barrier()

    def _drain_scatters():
        for b in range(B):
            pltpu.make_async_copy(
                zeros_hbm.at[pl.ds(0, CH)], bufs.at[0], ssem
            ).wait()

    @pl.loop(0, ROUNDS)
    def _(r):
        base0 = r * (NSETS * B)
        for half in range(NSETS):
            base = base0 + half * B
            descs = []
            for b in range(B):
                i = half * B + b
                descs.append(
                    pltpu.async_copy(gsh.at[srcv.at[base + b]], bufs.at[i], gsem)
                )
            if half == 0:
                @pl.when(r > 0)
                def _():
                    _drain_scatters()
            else:
                _drain_scatters()
            for d in descs:
                d.wait()
            for b in range(B):
                i = half * B + b
                pltpu.async_copy(bufs.at[i], acc.at[dstv.at[base + b]], ssem, add=True)

    _drain_scatters()
    plsc.subcore_barrier()
    pltpu.sync_copy(acc.at[pl.ds(s * RPT, RPT)], stage)
    pltpu.sync_copy(stage, out_hbm.at[c, pl.ds(s * RPT, RPT)])


_msg_call = pl.kernel(
    _msg_body,
    out_type=jax.ShapeDtypeStruct((NC, N_PAD, DH), jnp.float32),
    mesh=_mesh,
    compiler_params=_sc_params,
    scratch_types=[
        pltpu.VMEM((K, CH), jnp.int32),
        pltpu.VMEM((K, CH), jnp.int32),
        pltpu.VMEM((NSETS * B, CH, DH), jnp.float32),
        pltpu.VMEM((RPT, DH), jnp.float32),
        pltpu.VMEM_SHARED((N_PAD, DH), jnp.float32),
        pltpu.VMEM_SHARED((N_PAD, DH), jnp.float32),
        pltpu.SemaphoreType.DMA,
        pltpu.SemaphoreType.DMA,
    ],
)


def _mm_body(x_ref, w_ref, o_ref):
    o_ref[...] = jnp.dot(x_ref[...], w_ref[...], preferred_element_type=jnp.float32)


_mm_call = pl.pallas_call(
    _mm_body, out_shape=jax.ShapeDtypeStruct((NPK, 128), jnp.float32)
)


def _dinv_g_body(dp_ref, h_ref, dinv_ref, g_ref):
    dinv = lax.rsqrt(dp_ref[0] + dp_ref[1] + 1.0)
    dinv_ref[...] = dinv
    g_ref[...] = h_ref[...] * dinv


_dinv_g_call = pl.pallas_call(
    _dinv_g_body,
    out_shape=[
        jax.ShapeDtypeStruct((NPK, 128), jnp.float32),
        jax.ShapeDtypeStruct((NPK, 128), jnp.float32),
    ],
)


def _mid_body(p_ref, g_ref, dinv_ref, b_ref, lng_ref, lnb_ref, w_ref, mavg_ref, o_ref):
    dinv = dinv_ref[...]
    conv = dinv * (p_ref[0] + p_ref[1] + g_ref[...]) + b_ref[...]
    h = jnp.maximum(conv, 0.0)
    mavg = mavg_ref[...]
    mean = jnp.dot(h, mavg, preferred_element_type=jnp.float32)
    msq = jnp.dot(h * h, mavg, preferred_element_type=jnp.float32)
    var = jnp.maximum(msq - mean * mean, 0.0)
    hn = (h - mean) * lax.rsqrt(var + 1e-5) * lng_ref[...] + lnb_ref[...]
    o_ref[...] = (
        jnp.dot(hn, w_ref[...], preferred_element_type=jnp.float32) * dinv
    )


_mid_call = pl.pallas_call(
    _mid_body, out_shape=jax.ShapeDtypeStruct((NPK, 128), jnp.float32)
)


def _final_body(p_ref, g_ref, dinv_ref, b_ref, w0_ref, b0_ref, w1_ref, b1_ref, o_ref):
    conv = dinv_ref[...] * (p_ref[0] + p_ref[1] + g_ref[...]) + b_ref[...]
    h = jnp.maximum(conv, 0.0)
    y = jnp.maximum(
        jnp.dot(h, w0_ref[...], preferred_element_type=jnp.float32) + b0_ref[...], 0.0
    )
    y = jnp.dot(y, w1_ref[...], preferred_element_type=jnp.float32) + b1_ref[...]
    m = jnp.concatenate(
        [
            jnp.broadcast_to(
                jnp.max(y[:, g * DOUT : (g + 1) * DOUT], axis=-1, keepdims=True),
                (NPK, DOUT),
            )
            for g in range(4)
        ],
        axis=-1,
    )
    z = y - m
    ez = jnp.exp(z)
    lse = jnp.concatenate(
        [
            jnp.broadcast_to(
                jnp.log(
                    jnp.sum(ez[:, g * DOUT : (g + 1) * DOUT], axis=-1, keepdims=True)
                ),
                (NPK, DOUT),
            )
            for g in range(4)
        ],
        axis=-1,
    )
    o_ref[...] = z - lse


_final_call = pl.pallas_call(
    _final_body, out_shape=jax.ShapeDtypeStruct((NPK, 4 * DOUT), jnp.float32)
)


def _bd4(w):
    return block_diag(w, w, w, w)


def _tile4(v):
    return jnp.tile(v, 4).reshape(1, -1)


@jax.jit
def kernel(
    x,
    edge_index,
    conv_W0,
    conv_b0,
    conv_W1,
    conv_b1,
    conv_W2,
    conv_b2,
    ln_g0,
    ln_b0,
    ln_g1,
    ln_b1,
    mlp_W0,
    mlp_b0,
    mlp_W1,
    mlp_b1,
):
    f32 = jnp.float32
    xp = jnp.concatenate([x.astype(f32), jnp.zeros((N_PAD - N, DIN), f32)], axis=0)
    xpk = xp.reshape(NPK, 4 * DIN)
    src = edge_index[0].astype(jnp.int32)
    dst = edge_index[1].astype(jnp.int32)
    pad = jnp.full((E_PAD - E,), N, jnp.int32)
    src2 = jnp.concatenate([src, pad]).reshape(NW * K, CH)
    dst2 = jnp.concatenate([dst, pad]).reshape(NW * K, CH)

    zeros_dh = jnp.zeros((N_PAD, DH), f32)
    ones_dh = jnp.ones((CH, DH), f32)
    mavg = _bd4(jnp.full((DH, DH), 1.0 / DH, f32))

    degp = _deg_call(dst2, ones_dh, zeros_dh)
    h0 = _mm_call(xpk, _bd4(conv_W0))
    dinv, g0 = _dinv_g_call(degp.reshape(NC, NPK, 128), h0)

    p = _msg_call(g0.reshape(N_PAD, DH), src2, dst2, zeros_dh)
    g1 = _mid_call(
        p.reshape(NC, NPK, 128), g0, dinv, _tile4(conv_b0), _tile4(ln_g0),
        _tile4(ln_b0), _bd4(conv_W1), mavg,
    )
    p = _msg_call(g1.reshape(N_PAD, DH), src2, dst2, zeros_dh)
    g2 = _mid_call(
        p.reshape(NC, NPK, 128), g1, dinv, _tile4(conv_b1), _tile4(ln_g1),
        _tile4(ln_b1), _bd4(conv_W2), mavg,
    )
    p = _msg_call(g2.reshape(N_PAD, DH), src2, dst2, zeros_dh)
    out = _final_call(
        p.reshape(NC, NPK, 128),
        g2,
        dinv,
        _tile4(conv_b2),
        _bd4(mlp_W0),
        _tile4(mlp_b0),
        _bd4(mlp_W1),
        _tile4(mlp_b1),
    )
    return out.reshape(N_PAD, DOUT)[:N]
